# Initial kernel scaffold; baseline (speedup 1.0000x reference)
import jax
import jax.numpy as jnp
from jax.experimental import pallas as pl


def kernel(vertices, dir0, ste0, dir1, ste1, w1, b1, dir2, ste2, w2, b2, dir3, ste3, w3, b3, dir4, ste4, w4, b4, g1, be1, g2, be2, g3, be3):
    raise NotImplementedError("write your pallas kernel here")


# speedup vs baseline: 4.9842x; 4.9842x over previous
import functools
import math

import jax
import jax.numpy as jnp
from jax.experimental import pallas as pl
from jax.experimental.pallas import tpu as pltpu

_S = 7
_B = 4
_NEG = float("-inf")


def _norm_dirs(dirp):
    n = jnp.sqrt(jnp.sum(dirp * dirp, axis=0, keepdims=True))
    return dirp / jnp.maximum(n, 1e-12)


def _group_sum(x, groups, width):
    out = x[:, 0:width]
    for s in range(1, groups):
        out = out + x[:, s * width:(s + 1) * width]
    return out


def _knn_body(K, p_ref, pt_ref, idx_ref):
    p = p_ref[...]
    pt = pt_ref[...]
    V = p.shape[0]
    inner = jnp.dot(p, pt, preferred_element_type=jnp.float32)
    quad_c = jnp.sum(p * p, axis=1, keepdims=True)
    quad_r = jnp.sum(pt * pt, axis=0, keepdims=True)
    d = -2.0 * inner + quad_r + quad_c
    colid = jax.lax.broadcasted_iota(jnp.int32, (V, V), 1)
    inf = jnp.float32(float("inf"))
    for n in range(K):
        m = jnp.min(d, axis=1, keepdims=True)
        j = jnp.min(jnp.where(d == m, colid, V), axis=1, keepdims=True)
        idx_ref[:, n:n + 1] = j
        d = jnp.where(colid == j, inf, d)


def _knn(P, PT, K):
    B, V, _ = P.shape
    return pl.pallas_call(
        functools.partial(_knn_body, K),
        grid=(B,),
        in_specs=[
            pl.BlockSpec((None, V, 8), lambda b: (b, 0, 0)),
            pl.BlockSpec((None, 8, V), lambda b: (b, 0, 0)),
        ],
        out_specs=pl.BlockSpec((None, V, 16), lambda b: (b, 0, 0)),
        out_shape=jax.ShapeDtypeStruct((B, V, 16), jnp.int32),
    )(P, PT)


def _split3(x):
    hi = x.astype(jnp.bfloat16)
    r1 = x - hi.astype(jnp.float32)
    mid = r1.astype(jnp.bfloat16)
    lo = (r1 - mid.astype(jnp.float32)).astype(jnp.bfloat16)
    return hi, mid, lo


def _gsel(G, x):
    Gb = G.astype(jnp.bfloat16)
    hi, mid, lo = _split3(x)
    o = jnp.dot(Gb, hi, preferred_element_type=jnp.float32)
    o = o + jnp.dot(Gb, mid, preferred_element_type=jnp.float32)
    return o + jnp.dot(Gb, lo, preferred_element_type=jnp.float32)


def _gselr(x, G):
    Gb = G.astype(jnp.bfloat16)
    hi, mid, lo = _split3(x)
    o = jnp.dot(hi, Gb, preferred_element_type=jnp.float32)
    o = o + jnp.dot(mid, Gb, preferred_element_type=jnp.float32)
    return o + jnp.dot(lo, Gb, preferred_element_type=jnp.float32)


def _dirnorm(p, G):
    nbr = _gsel(G, p)
    dvec = nbr - p
    nrm = jnp.sqrt(jnp.sum(dvec * dvec, axis=1, keepdims=True))
    return dvec / jnp.maximum(nrm, 1e-12)


def _surface_body(p_ref, idx_ref, dirp_ref, steT_ref, out_ref):
    B, V = p_ref.shape[0], p_ref.shape[1]
    sdir = _norm_dirs(dirp_ref[...])
    steT = steT_ref[...]
    colid = jax.lax.broadcasted_iota(jnp.int32, (V, V), 1)
    for b in range(B):
        p = p_ref[b]
        fste = jnp.dot(p, steT, preferred_element_type=jnp.float32)
        acc = jnp.full((V, sdir.shape[1]), _NEG, jnp.float32)
        for n in range(1, 11):
            j = idx_ref[b][:, n:n + 1]
            G = (colid == j).astype(jnp.float32)
            dn = _dirnorm(p, G)
            theta = jnp.maximum(
                jnp.dot(dn, sdir, preferred_element_type=jnp.float32), 0.0)
            acc = jnp.maximum(acc, theta)
        forl = _group_sum(acc, _S, 128)
        out_ref[b] = jnp.maximum(fste + forl, 0.0)


def _surface(P, idx, dirp, steT):
    B, V, _ = P.shape
    return pl.pallas_call(
        _surface_body,
        out_shape=jax.ShapeDtypeStruct((B, V, 128), jnp.float32),
    )(P, idx, dirp, steT)


def _hs_body(N, Cout, bn, finalmax,
             f_ref, p_ref, idx_ref, dirp_ref, steT_ref, w_ref, b_ref,
             g_ref, be_ref, out_ref, pre_ref):
    B, V = f_ref.shape[0], f_ref.shape[1]
    sdir = _norm_dirs(dirp_ref[...])
    steT = steT_ref[...]
    w = w_ref[...]
    brow = b_ref[...]
    colid = jax.lax.broadcasted_iota(jnp.int32, (V, V), 1)
    s1 = jnp.zeros((1, Cout), jnp.float32)
    s2 = jnp.zeros((1, Cout), jnp.float32)
    for b in range(B):
        f = f_ref[b]
        p = p_ref[b]
        fste = jnp.dot(f, steT, preferred_element_type=jnp.float32)
        fout = jnp.dot(f, w, preferred_element_type=jnp.float32) + brow
        center = fout[:, :Cout]
        support = fout[:, Cout:]
        acc = jnp.full((V, _S * Cout), _NEG, jnp.float32)
        for n in range(1, N + 1):
            j = idx_ref[b][:, n:n + 1]
            G = (colid == j).astype(jnp.float32)
            dn = _dirnorm(p, G)
            theta = jnp.maximum(
                jnp.dot(dn, sdir, preferred_element_type=jnp.float32), 0.0)
            snbr = _gsel(G, support)
            acc = jnp.maximum(acc, theta * snbr)
        act = _group_sum(acc, _S, Cout)
        pre = fste + center + act
        if bn:
            pre_ref[b] = pre
            s1 = s1 + jnp.sum(pre, axis=0, keepdims=True)
            s2 = s2 + jnp.sum(pre * pre, axis=0, keepdims=True)
        elif finalmax:
            out_ref[b:b + 1, :] = jnp.max(pre, axis=0, keepdims=True)
        else:
            out_ref[b] = pre
    if bn:
        cnt = jnp.float32(B * V)
        mean = s1 / cnt
        var = s2 / cnt - mean * mean
        scale = g_ref[...] / jnp.sqrt(var + 1e-5)
        shift = be_ref[...] - mean * scale
        for b in range(B):
            out_ref[b] = jnp.maximum(pre_ref[b] * scale + shift, 0.0)


def _hs(fmap, P, idx, N, Cout, dirp, steT, w, brow, grow, berow,
        bn=True, finalmax=False):
    B, V, _ = fmap.shape
    out_shape = ((B, Cout) if finalmax else (B, V, Cout))
    return pl.pallas_call(
        functools.partial(_hs_body, N, Cout, bn, finalmax),
        out_shape=jax.ShapeDtypeStruct(out_shape, jnp.float32),
        scratch_shapes=[pltpu.VMEM((B, V, Cout), jnp.float32)],
    )(fmap, P, idx, dirp, steT, w, brow, grow, berow)


def _pool_body(Vp, f_ref, p_ref, pt_ref, idx_ref, permc_ref, permr_ref,
               fo_ref, vo_ref, vto_ref):
    B, V, C = f_ref.shape
    colid = jax.lax.broadcasted_iota(jnp.int32, (V, V), 1)
    colid_p = jax.lax.broadcasted_iota(jnp.int32, (Vp, V), 1)
    rowid = jax.lax.broadcasted_iota(jnp.int32, (V, Vp), 0)
    sel = (colid_p == permc_ref[...]).astype(jnp.float32)
    selT = (rowid == permr_ref[...]).astype(jnp.float32)
    for b in range(B):
        f = f_ref[b]
        pooled = jnp.full((V, C), _NEG, jnp.float32)
        for n in range(1, 5):
            j = idx_ref[b][:, n:n + 1]
            G = (colid == j).astype(jnp.float32)
            pooled = jnp.maximum(pooled, _gsel(G, f))
        fo_ref[b] = _gsel(sel, pooled)
        vo_ref[b] = _gsel(sel, p_ref[b])
        vto_ref[b] = _gselr(pt_ref[b], selT)


def _pool(fmap, P, PT, idx, permc, permr):
    B, V, C = fmap.shape
    Vp = permc.shape[0]
    return pl.pallas_call(
        functools.partial(_pool_body, Vp),
        out_shape=(
            jax.ShapeDtypeStruct((B, Vp, C), jnp.float32),
            jax.ShapeDtypeStruct((B, Vp, 8), jnp.float32),
            jax.ShapeDtypeStruct((B, 8, Vp), jnp.float32),
        ),
    )(fmap, P, PT, idx, permc, permr)


def _pad8(m):
    return jnp.pad(m, ((0, 8 - m.shape[0]), (0, 0)))


def kernel(vertices, dir0, ste0, dir1, ste1, w1, b1, dir2, ste2, w2, b2,
           dir3, ste3, w3, b3, dir4, ste4, w4, b4, g1, be1, g2, be2, g3, be3):
    B, V0, _ = vertices.shape
    P = jnp.pad(vertices, ((0, 0), (0, 0), (0, 5)))
    PT = jnp.transpose(P, (0, 2, 1))

    perm1 = jax.random.permutation(jax.random.key(1235), V0)[:V0 // 4]
    perm2 = jax.random.permutation(jax.random.key(1236), V0 // 4)[:V0 // 16]
    p1c = perm1.astype(jnp.int32)[:, None]
    p1r = perm1.astype(jnp.int32)[None, :]
    p2c = perm2.astype(jnp.int32)[:, None]
    p2r = perm2.astype(jnp.int32)[None, :]

    row = lambda x: x[None, :]

    idx0 = _knn(P, PT, 11)
    fm0 = _surface(P, idx0, _pad8(dir0), _pad8(ste0.T))
    fm1 = _hs(fm0, P, idx0, 10, 128, _pad8(dir1), ste1.T, w1, row(b1),
              row(g1), row(be1))
    f1, v1P, v1PT = _pool(fm1, P, PT, idx0, p1c, p1r)

    idx1 = _knn(v1P, v1PT, 11)
    fm2 = _hs(f1, v1P, idx1, 10, 256, _pad8(dir2), ste2.T, w2, row(b2),
              row(g2), row(be2))
    fm3 = _hs(fm2, v1P, idx1, 10, 256, _pad8(dir3), ste3.T, w3, row(b3),
              row(g3), row(be3))
    f2, v2P, v2PT = _pool(fm3, v1P, v1PT, idx1, p2c, p2r)

    idx2 = _knn(v2P, v2PT, 9)
    return _hs(f2, v2P, idx2, 8, 512, _pad8(dir4), ste4.T, w4, row(b4),
               row(b4), row(b4), bn=False, finalmax=True)


# speedup vs baseline: 5.4810x; 1.0997x over previous
import functools
import math

import jax
import jax.numpy as jnp
from jax.experimental import pallas as pl
from jax.experimental.pallas import tpu as pltpu

_S = 7
_B = 4
_NEG = float("-inf")


def _norm_dirs(dirp):
    n = jnp.sqrt(jnp.sum(dirp * dirp, axis=0, keepdims=True))
    return dirp / jnp.maximum(n, 1e-12)


def _group_sum(x, groups, width):
    out = x[:, 0:width]
    for s in range(1, groups):
        out = out + x[:, s * width:(s + 1) * width]
    return out


def _knn_body(K, p_ref, pt_ref, idx_ref):
    p = p_ref[...]
    pt = pt_ref[...]
    V = p.shape[0]
    inner = jnp.dot(p, pt, preferred_element_type=jnp.float32)
    quad_c = jnp.sum(p * p, axis=1, keepdims=True)
    quad_r = jnp.sum(pt * pt, axis=0, keepdims=True)
    d = -2.0 * inner + quad_r + quad_c
    colid = jax.lax.broadcasted_iota(jnp.int32, (V, V), 1)
    inf = jnp.float32(float("inf"))
    for n in range(K):
        m = jnp.min(d, axis=1, keepdims=True)
        j = jnp.min(jnp.where(d == m, colid, V), axis=1, keepdims=True)
        idx_ref[:, n:n + 1] = j
        d = jnp.where(colid == j, inf, d)


def _knn(P, PT, K):
    B, V, _ = P.shape
    return pl.pallas_call(
        functools.partial(_knn_body, K),
        grid=(B,),
        in_specs=[
            pl.BlockSpec((None, V, 8), lambda b: (b, 0, 0)),
            pl.BlockSpec((None, 8, V), lambda b: (b, 0, 0)),
        ],
        out_specs=pl.BlockSpec((None, V, 16), lambda b: (b, 0, 0)),
        out_shape=jax.ShapeDtypeStruct((B, V, 16), jnp.int32),
    )(P, PT)


def _split3(x):
    hi = x.astype(jnp.bfloat16)
    r1 = x - hi.astype(jnp.float32)
    mid = r1.astype(jnp.bfloat16)
    lo = (r1 - mid.astype(jnp.float32)).astype(jnp.bfloat16)
    return hi, mid, lo


def _gsel(G, x, passes=3):
    Gb = G.astype(jnp.bfloat16)
    hi, mid, lo = _split3(x)
    o = jnp.dot(Gb, hi, preferred_element_type=jnp.float32)
    o = o + jnp.dot(Gb, mid, preferred_element_type=jnp.float32)
    if passes == 3:
        o = o + jnp.dot(Gb, lo, preferred_element_type=jnp.float32)
    return o


def _gselr(x, G):
    Gb = G.astype(jnp.bfloat16)
    hi, mid, lo = _split3(x)
    o = jnp.dot(hi, Gb, preferred_element_type=jnp.float32)
    o = o + jnp.dot(mid, Gb, preferred_element_type=jnp.float32)
    return o + jnp.dot(lo, Gb, preferred_element_type=jnp.float32)


def _dirnorm(p, G):
    nbr = _gsel(G, p)
    dvec = nbr - p
    nrm = jnp.sqrt(jnp.sum(dvec * dvec, axis=1, keepdims=True))
    return dvec / jnp.maximum(nrm, 1e-12)


def _surface_body(p_ref, idx_ref, dirp_ref, steT_ref, out_ref):
    B, V = p_ref.shape[0], p_ref.shape[1]
    sdir = _norm_dirs(dirp_ref[...])
    steT = steT_ref[...]
    colid = jax.lax.broadcasted_iota(jnp.int32, (V, V), 1)
    for b in range(B):
        p = p_ref[b]
        fste = jnp.dot(p, steT, preferred_element_type=jnp.float32)
        acc = jnp.full((V, sdir.shape[1]), _NEG, jnp.float32)
        for n in range(1, 11):
            j = idx_ref[b][:, n:n + 1]
            G = (colid == j).astype(jnp.float32)
            dn = _dirnorm(p, G)
            theta = jnp.maximum(
                jnp.dot(dn, sdir, preferred_element_type=jnp.float32), 0.0)
            acc = jnp.maximum(acc, theta)
        forl = _group_sum(acc, _S, 128)
        out_ref[b] = jnp.maximum(fste + forl, 0.0)


def _surface(P, idx, dirp, steT):
    B, V, _ = P.shape
    return pl.pallas_call(
        _surface_body,
        out_shape=jax.ShapeDtypeStruct((B, V, 128), jnp.float32),
    )(P, idx, dirp, steT)


def _hs_body(N, Cout, bn, finalmax,
             f_ref, p_ref, idx_ref, dirp_ref, steT_ref, w_ref, b_ref,
             g_ref, be_ref, out_ref, pre_ref):
    B, V = f_ref.shape[0], f_ref.shape[1]
    sdir = _norm_dirs(dirp_ref[...])
    steT = steT_ref[...]
    w = w_ref[...]
    brow = b_ref[...]
    colid = jax.lax.broadcasted_iota(jnp.int32, (V, V), 1)
    s1 = jnp.zeros((1, Cout), jnp.float32)
    s2 = jnp.zeros((1, Cout), jnp.float32)
    for b in range(B):
        f = f_ref[b]
        p = p_ref[b]
        fste = jnp.dot(f, steT, preferred_element_type=jnp.float32)
        fout = jnp.dot(f, w, preferred_element_type=jnp.float32) + brow
        center = fout[:, :Cout]
        support = fout[:, Cout:]
        acc = jnp.full((V, _S * Cout), _NEG, jnp.float32)
        for n in range(1, N + 1):
            j = idx_ref[b][:, n:n + 1]
            G = (colid == j).astype(jnp.float32)
            dn = _dirnorm(p, G)
            theta = jnp.maximum(
                jnp.dot(dn, sdir, preferred_element_type=jnp.float32), 0.0)
            snbr = _gsel(G, support, passes=2)
            acc = jnp.maximum(acc, theta * snbr)
        act = _group_sum(acc, _S, Cout)
        pre = fste + center + act
        if bn:
            pre_ref[b] = pre
            s1 = s1 + jnp.sum(pre, axis=0, keepdims=True)
            s2 = s2 + jnp.sum(pre * pre, axis=0, keepdims=True)
        elif finalmax:
            out_ref[b:b + 1, :] = jnp.max(pre, axis=0, keepdims=True)
        else:
            out_ref[b] = pre
    if bn:
        cnt = jnp.float32(B * V)
        mean = s1 / cnt
        var = s2 / cnt - mean * mean
        scale = g_ref[...] / jnp.sqrt(var + 1e-5)
        shift = be_ref[...] - mean * scale
        for b in range(B):
            out_ref[b] = jnp.maximum(pre_ref[b] * scale + shift, 0.0)


def _hs(fmap, P, idx, N, Cout, dirp, steT, w, brow, grow, berow,
        bn=True, finalmax=False):
    B, V, _ = fmap.shape
    out_shape = ((B, Cout) if finalmax else (B, V, Cout))
    return pl.pallas_call(
        functools.partial(_hs_body, N, Cout, bn, finalmax),
        out_shape=jax.ShapeDtypeStruct(out_shape, jnp.float32),
        scratch_shapes=[pltpu.VMEM((B, V, Cout), jnp.float32)],
    )(fmap, P, idx, dirp, steT, w, brow, grow, berow)


def _pool_body(Vp, f_ref, p_ref, pt_ref, idx_ref, permc_ref, permr_ref,
               fo_ref, vo_ref, vto_ref):
    B, V, C = f_ref.shape
    colid = jax.lax.broadcasted_iota(jnp.int32, (V, V), 1)
    colid_p = jax.lax.broadcasted_iota(jnp.int32, (Vp, V), 1)
    rowid = jax.lax.broadcasted_iota(jnp.int32, (V, Vp), 0)
    sel = (colid_p == permc_ref[...]).astype(jnp.float32)
    selT = (rowid == permr_ref[...]).astype(jnp.float32)
    for b in range(B):
        f = f_ref[b]
        pooled = jnp.full((V, C), _NEG, jnp.float32)
        for n in range(1, 5):
            j = idx_ref[b][:, n:n + 1]
            G = (colid == j).astype(jnp.float32)
            pooled = jnp.maximum(pooled, _gsel(G, f, passes=2))
        fo_ref[b] = _gsel(sel, pooled, passes=2)
        vo_ref[b] = _gsel(sel, p_ref[b])
        vto_ref[b] = _gselr(pt_ref[b], selT)


def _pool(fmap, P, PT, idx, permc, permr):
    B, V, C = fmap.shape
    Vp = permc.shape[0]
    return pl.pallas_call(
        functools.partial(_pool_body, Vp),
        out_shape=(
            jax.ShapeDtypeStruct((B, Vp, C), jnp.float32),
            jax.ShapeDtypeStruct((B, Vp, 8), jnp.float32),
            jax.ShapeDtypeStruct((B, 8, Vp), jnp.float32),
        ),
    )(fmap, P, PT, idx, permc, permr)


def _pad8(m):
    return jnp.pad(m, ((0, 8 - m.shape[0]), (0, 0)))


def kernel(vertices, dir0, ste0, dir1, ste1, w1, b1, dir2, ste2, w2, b2,
           dir3, ste3, w3, b3, dir4, ste4, w4, b4, g1, be1, g2, be2, g3, be3):
    B, V0, _ = vertices.shape
    P = jnp.pad(vertices, ((0, 0), (0, 0), (0, 5)))
    PT = jnp.transpose(P, (0, 2, 1))

    perm1 = jax.random.permutation(jax.random.key(1235), V0)[:V0 // 4]
    perm2 = jax.random.permutation(jax.random.key(1236), V0 // 4)[:V0 // 16]
    p1c = perm1.astype(jnp.int32)[:, None]
    p1r = perm1.astype(jnp.int32)[None, :]
    p2c = perm2.astype(jnp.int32)[:, None]
    p2r = perm2.astype(jnp.int32)[None, :]

    row = lambda x: x[None, :]

    idx0 = _knn(P, PT, 11)
    fm0 = _surface(P, idx0, _pad8(dir0), _pad8(ste0.T))
    fm1 = _hs(fm0, P, idx0, 10, 128, _pad8(dir1), ste1.T, w1, row(b1),
              row(g1), row(be1))
    f1, v1P, v1PT = _pool(fm1, P, PT, idx0, p1c, p1r)

    idx1 = _knn(v1P, v1PT, 11)
    fm2 = _hs(f1, v1P, idx1, 10, 256, _pad8(dir2), ste2.T, w2, row(b2),
              row(g2), row(be2))
    fm3 = _hs(fm2, v1P, idx1, 10, 256, _pad8(dir3), ste3.T, w3, row(b3),
              row(g3), row(be3))
    f2, v2P, v2PT = _pool(fm3, v1P, v1PT, idx1, p2c, p2r)

    idx2 = _knn(v2P, v2PT, 9)
    return _hs(f2, v2P, idx2, 8, 512, _pad8(dir4), ste4.T, w4, row(b4),
               row(b4), row(b4), bn=False, finalmax=True)
